# baseline (speedup 1.0000x reference)
import functools

import jax
import jax.numpy as jnp
from jax import lax
from jax.experimental import pallas as pl
from jax.experimental.pallas import tpu as pltpu
from jax.experimental.pallas import tpu_sc as plsc

_NC = 2
_NS = 16
_NW = _NC * _NS

_S = 4
_P = 20
_NBUF = 4


def _embed_gather(idx2, table_pad, b, d):
    per_w = idx2.shape[1]
    dp = table_pad.shape[1]
    sg = _S * _P
    n_chunks = per_w // sg
    mesh = plsc.VectorSubcoreMesh(core_axis_name="c", subcore_axis_name="s")

    @functools.partial(
        pl.kernel,
        mesh=mesh,
        compiler_params=pltpu.CompilerParams(use_tc_tiling_on_sc=True),
        out_type=jax.ShapeDtypeStruct((b, _P, d), jnp.float32),
        scratch_types=[
            pltpu.VMEM((per_w,), jnp.int32),
            *[pltpu.VMEM((sg, dp), jnp.float32) for _ in range(_NBUF)],
            *[pltpu.VMEM((_S, _P, d), jnp.float32) for _ in range(_NBUF)],
            *[pltpu.SemaphoreType.DMA for _ in range(3 * _NBUF)],
        ],
    )
    def k(idx_hbm, table_hbm, out_hbm, idx_v, *bufs_sems):
        bufs = bufs_sems[:_NBUF]
        buf2s = bufs_sems[_NBUF:2 * _NBUF]
        gsems = bufs_sems[2 * _NBUF:3 * _NBUF]
        csems = bufs_sems[3 * _NBUF:4 * _NBUF]
        osems = bufs_sems[4 * _NBUF:]
        wid = lax.axis_index("s") * _NC + lax.axis_index("c")
        sent_base = wid * (per_w // _P)
        pltpu.sync_copy(idx_hbm.at[wid], idx_v)

        def issue(c, bi):
            pltpu.async_copy(
                table_hbm.at[idx_v.at[pl.ds(c * sg, sg)]], bufs[bi], gsems[bi])

        for bi in range(_NBUF):
            issue(bi, bi)

        def body(cp, carry):
            c0 = cp * _NBUF
            for bi in range(_NBUF):
                c = c0 + bi
                pltpu.make_async_copy(
                    table_hbm.at[pl.ds(0, sg)], bufs[bi], gsems[bi]).wait()
                @pl.when(cp > 0)
                def _():
                    pltpu.make_async_copy(
                        buf2s[bi], out_hbm.at[pl.ds(0, _S)], osems[bi]).wait()
                for r in range(sg):
                    ls, t = r // _P, r % _P
                    for o in (0, 16, 32, d - 16):
                        buf2s[bi][ls, t, pl.ds(o, 16)] = (
                            bufs[bi][r, pl.ds(o, 16)])
                pltpu.async_copy(
                    buf2s[bi], out_hbm.at[pl.ds(sent_base + c * _S, _S)],
                    osems[bi])
                c_next = c + _NBUF

                @pl.when(c_next < n_chunks)
                def _():
                    issue(c_next, bi)

            return carry

        lax.fori_loop(0, n_chunks // _NBUF, body, 0)
        for bi in range(_NBUF):
            pltpu.make_async_copy(
                buf2s[bi], out_hbm.at[pl.ds(0, _S)], osems[bi]).wait()

    return k(idx2, table_pad)


def kernel(indices, table):
    b, p = indices.shape
    v, d = table.shape
    n = b * p
    per_w = n // _NW
    idx2 = indices.astype(jnp.int32).reshape(_NW, per_w)
    table_pad = jnp.pad(table, ((0, 0), (0, 128 - d)))
    return _embed_gather(idx2, table_pad, b, d)


# speedup vs baseline: 1.4665x; 1.4665x over previous
import functools

import jax
import jax.numpy as jnp
from jax import lax
from jax.experimental import pallas as pl
from jax.experimental.pallas import tpu as pltpu
from jax.experimental.pallas import tpu_sc as plsc

_NC = 2
_NS = 16
_NW = _NC * _NS

_S = 4
_P = 20
_NBUF = 4


def _embed_gather(idx2, table_pad, b, d):
    per_w = idx2.shape[1]
    dp = table_pad.shape[1]
    sg = _S * _P
    n_chunks = per_w // sg
    mesh = plsc.VectorSubcoreMesh(core_axis_name="c", subcore_axis_name="s")

    @functools.partial(
        pl.kernel,
        mesh=mesh,
        compiler_params=pltpu.CompilerParams(use_tc_tiling_on_sc=True),
        out_type=jax.ShapeDtypeStruct((b, _P, d), jnp.float32),
        scratch_types=[
            pltpu.VMEM((per_w,), jnp.int32),
            *[pltpu.VMEM((sg, dp), jnp.float32) for _ in range(_NBUF)],
            *[pltpu.VMEM((_S, _P, d), jnp.float32) for _ in range(_NBUF)],
            *[pltpu.SemaphoreType.DMA for _ in range(3 * _NBUF)],
        ],
    )
    def k(idx_hbm, table_hbm, out_hbm, idx_v, *bufs_sems):
        bufs = bufs_sems[:_NBUF]
        buf2s = bufs_sems[_NBUF:2 * _NBUF]
        gsems = bufs_sems[2 * _NBUF:3 * _NBUF]
        csems = bufs_sems[3 * _NBUF:4 * _NBUF]
        osems = bufs_sems[4 * _NBUF:]
        wid = lax.axis_index("s") * _NC + lax.axis_index("c")
        sent_base = wid * (per_w // _P)
        pltpu.sync_copy(idx_hbm.at[wid], idx_v)

        def issue(c, bi):
            pltpu.async_copy(
                table_hbm.at[idx_v.at[pl.ds(c * sg, sg)]], bufs[bi], gsems[bi])

        for bi in range(_NBUF):
            issue(bi, bi)

        def body(cp, carry):
            c0 = cp * _NBUF
            for bi in range(_NBUF):
                c = c0 + bi
                pltpu.make_async_copy(
                    table_hbm.at[pl.ds(0, sg)], bufs[bi], gsems[bi]).wait()
                @pl.when(cp > 0)
                def _():
                    pltpu.make_async_copy(
                        buf2s[bi], out_hbm.at[pl.ds(0, _S)], osems[bi]).wait()
                for r in range(sg):
                    ls, t = r // _P, r % _P
                    for o in (0, 16, 32, d - 16):
                        buf2s[bi][ls, t, pl.ds(o, 16)] = (
                            bufs[bi][r, pl.ds(o, 16)])
                pltpu.async_copy(
                    buf2s[bi], out_hbm.at[pl.ds(sent_base + c * _S, _S)],
                    osems[bi])
                c_next = c + _NBUF

                @pl.when(c_next < n_chunks)
                def _():
                    issue(c_next, bi)

            return carry

        lax.fori_loop(0, n_chunks // _NBUF, body, 0)
        for bi in range(_NBUF):
            pltpu.make_async_copy(
                buf2s[bi], out_hbm.at[pl.ds(0, _S)], osems[bi]).wait()

    return k(idx2, table_pad)


def _pad_rows_tc(table):
    v, d = table.shape
    blk = 4000
    assert v % blk == 0

    def body(in_ref, out_ref):
        out_ref[:, :d] = in_ref[...]

    return pl.pallas_call(
        body,
        grid=(v // blk,),
        in_specs=[pl.BlockSpec((blk, d), lambda i: (i, 0))],
        out_specs=pl.BlockSpec((blk, 128), lambda i: (i, 0)),
        out_shape=jax.ShapeDtypeStruct((v, 128), jnp.float32),
    )(table)


def kernel(indices, table):
    b, p = indices.shape
    v, d = table.shape
    n = b * p
    per_w = n // _NW
    idx2 = indices.astype(jnp.int32).reshape(_NW, per_w)
    table_pad = _pad_rows_tc(table)
    return _embed_gather(idx2, table_pad, b, d)


# speedup vs baseline: 1.5083x; 1.0285x over previous
import functools

import jax
import jax.numpy as jnp
from jax import lax
from jax.experimental import pallas as pl
from jax.experimental.pallas import tpu as pltpu
from jax.experimental.pallas import tpu_sc as plsc

_NC = 2
_NS = 16
_NW = _NC * _NS

_S = 4
_P = 20
_NBUF = 4


def _embed_gather(idx2, table_pad, b, d):
    per_w = idx2.shape[1]
    dp = table_pad.shape[1]
    sg = _S * _P
    n_chunks = per_w // sg
    mesh = plsc.VectorSubcoreMesh(core_axis_name="c", subcore_axis_name="s")

    @functools.partial(
        pl.kernel,
        mesh=mesh,
        compiler_params=pltpu.CompilerParams(use_tc_tiling_on_sc=True),
        out_type=jax.ShapeDtypeStruct((b, _P, d), jnp.float32),
        scratch_types=[
            pltpu.VMEM((per_w,), jnp.int32),
            *[pltpu.VMEM((sg, dp), jnp.float32) for _ in range(_NBUF)],
            *[pltpu.VMEM((_S, _P, d), jnp.float32) for _ in range(_NBUF)],
            *[pltpu.SemaphoreType.DMA for _ in range(3 * _NBUF)],
        ],
    )
    def k(idx_hbm, table_hbm, out_hbm, idx_v, *bufs_sems):
        bufs = bufs_sems[:_NBUF]
        buf2s = bufs_sems[_NBUF:2 * _NBUF]
        gsems = bufs_sems[2 * _NBUF:3 * _NBUF]
        csems = bufs_sems[3 * _NBUF:4 * _NBUF]
        osems = bufs_sems[4 * _NBUF:]
        wid = lax.axis_index("s") * _NC + lax.axis_index("c")
        sent_base = wid * (per_w // _P)
        pltpu.sync_copy(idx_hbm.at[wid], idx_v)

        def issue(c, bi):
            pltpu.async_copy(
                table_hbm.at[idx_v.at[pl.ds(c * sg, sg)]], bufs[bi], gsems[bi])

        for bi in range(_NBUF):
            issue(bi, bi)

        def body(cp, carry):
            c0 = cp * _NBUF
            for bi in range(_NBUF):
                c = c0 + bi
                pltpu.make_async_copy(
                    table_hbm.at[pl.ds(0, sg)], bufs[bi], gsems[bi]).wait()
                @pl.when(cp > 0)
                def _():
                    pltpu.make_async_copy(
                        buf2s[bi], out_hbm.at[pl.ds(0, _S)], osems[bi]).wait()
                for r in range(sg):
                    ls, t = r // _P, r % _P
                    for o in (0, 16, 32, d - 16):
                        buf2s[bi][ls, t, pl.ds(o, 16)] = (
                            bufs[bi][r, pl.ds(o, 16)])
                pltpu.async_copy(
                    buf2s[bi], out_hbm.at[pl.ds(sent_base + c * _S, _S)],
                    osems[bi])
                c_next = c + _NBUF

                @pl.when(c_next < n_chunks)
                def _():
                    issue(c_next, bi)

            return carry

        lax.fori_loop(0, n_chunks // _NBUF, body, 0)
        for bi in range(_NBUF):
            pltpu.make_async_copy(
                buf2s[bi], out_hbm.at[pl.ds(0, _S)], osems[bi]).wait()

    return k(idx2, table_pad)


def _pad_rows_tc(table):
    v, d = table.shape
    blk = 8000
    assert v % blk == 0

    def body(in_ref, out_ref):
        out_ref[...] = jnp.pad(in_ref[...], ((0, 0), (0, 128 - d)))

    return pl.pallas_call(
        body,
        grid=(v // blk,),
        in_specs=[pl.BlockSpec((blk, d), lambda i: (i, 0))],
        out_specs=pl.BlockSpec((blk, 128), lambda i: (i, 0)),
        out_shape=jax.ShapeDtypeStruct((v, 128), jnp.float32),
    )(table)


def kernel(indices, table):
    b, p = indices.shape
    v, d = table.shape
    n = b * p
    per_w = n // _NW
    idx2 = indices.astype(jnp.int32).reshape(_NW, per_w)
    table_pad = _pad_rows_tc(table)
    return _embed_gather(idx2, table_pad, b, d)


# speedup vs baseline: 2.2807x; 1.5122x over previous
import functools

import jax
import jax.numpy as jnp
from jax import lax
from jax.experimental import pallas as pl
from jax.experimental.pallas import tpu as pltpu
from jax.experimental.pallas import tpu_sc as plsc

_NC = 2
_NS = 16
_NW = _NC * _NS

_S = 8
_P = 20
_NBUF = 4


def _embed_gather(idx2, table, b):
    per_w = idx2.shape[1]
    d = table.shape[1]
    sg = _S * _P
    n_chunks = per_w // sg
    mesh = plsc.VectorSubcoreMesh(core_axis_name="c", subcore_axis_name="s")

    @functools.partial(
        pl.kernel,
        mesh=mesh,
        compiler_params=pltpu.CompilerParams(use_tc_tiling_on_sc=True),
        out_type=jax.ShapeDtypeStruct((b, _P, d), jnp.float32),
        scratch_types=[
            pltpu.VMEM((per_w,), jnp.int32),
            *[pltpu.VMEM((_S, _P, d), jnp.float32) for _ in range(_NBUF)],
            *[pltpu.SemaphoreType.DMA for _ in range(2 * _NBUF)],
        ],
    )
    def k(idx_hbm, table_hbm, out_hbm, idx_v, *bufs_sems):
        bufs = bufs_sems[:_NBUF]
        gsems = bufs_sems[_NBUF:2 * _NBUF]
        osems = bufs_sems[2 * _NBUF:]
        wid = lax.axis_index("s") * _NC + lax.axis_index("c")
        sent_base = wid * (per_w // _P)
        pltpu.sync_copy(idx_hbm.at[wid], idx_v)

        def issue(c, bi):
            for g in range(sg // 16):
                vec = idx_v[pl.ds(c * sg + g * 16, 16)]
                for j in range(16):
                    r = g * 16 + j
                    pltpu.async_copy(
                        table_hbm.at[vec[j]], bufs[bi].at[r // _P, r % _P],
                        gsems[bi])

        for bi in range(_NBUF):
            issue(bi, bi)

        def body(cp, carry):
            c0 = cp * _NBUF
            for bi in range(_NBUF):
                pltpu.make_async_copy(
                    out_hbm.at[pl.ds(0, _S)], bufs[bi], gsems[bi]).wait()
                pltpu.async_copy(
                    bufs[bi],
                    out_hbm.at[pl.ds(sent_base + (c0 + bi) * _S, _S)],
                    osems[bi])
            for bi in range(_NBUF):
                c_next = c0 + _NBUF + bi

                @pl.when(c_next < n_chunks)
                def _():
                    pltpu.make_async_copy(
                        bufs[bi], out_hbm.at[pl.ds(0, _S)], osems[bi]).wait()
                    issue(c_next, bi)

            return carry

        lax.fori_loop(0, n_chunks // _NBUF, body, 0)
        for bi in range(_NBUF):
            pltpu.make_async_copy(
                bufs[bi], out_hbm.at[pl.ds(0, _S)], osems[bi]).wait()

    return k(idx2, table)


def kernel(indices, table):
    b, p = indices.shape
    n = b * p
    per_w = n // _NW
    idx2 = indices.astype(jnp.int32).reshape(_NW, per_w)
    return _embed_gather(idx2, table, b)
